# Initial kernel scaffold; baseline (speedup 1.0000x reference)
import jax
import jax.numpy as jnp
from jax.experimental import pallas as pl


def kernel(x, edge_idx, TT, weight, bias_param):
    raise NotImplementedError("write your pallas kernel here")


# speedup vs baseline: 28.7220x; 28.7220x over previous
import functools

import jax
import jax.numpy as jnp
from jax import lax
from jax.experimental import pallas as pl
from jax.experimental.pallas import tpu as pltpu
from jax.experimental.pallas import tpu_sc as plsc

_NC = 2
_NS = 16
_L = 16

_GATHER_DNUMS = lax.GatherDimensionNumbers(
    offset_dims=(), collapsed_slice_dims=(0,), start_index_map=(0,))


def _splat(v, i):
    lane = jnp.full((_L, 1), i, jnp.int32)
    return lax.gather(v, lane, _GATHER_DNUMS, slice_sizes=(1,),
                      mode=lax.GatherScatterMode.PROMISE_IN_BOUNDS)


def _matmul_body(x_ref, w_ref, o_ref):
    o_ref[...] = jnp.dot(x_ref[...], w_ref[...],
                         preferred_element_type=jnp.float32)


def _support_matmul(x, wf):
    n, d_in = x.shape
    dk = wf.shape[1]
    blk = 2000
    return pl.pallas_call(
        _matmul_body,
        grid=(n // blk,),
        in_specs=[pl.BlockSpec((blk, d_in), lambda i: (i, 0)),
                  pl.BlockSpec((d_in, dk), lambda i: (0, 0))],
        out_specs=pl.BlockSpec((blk, dk), lambda i: (i, 0)),
        out_shape=jax.ShapeDtypeStruct((n, dk), jnp.float32),
    )(x, wf)


def _combine_body(p_ref, b_ref, o_ref):
    o_ref[...] = p_ref[0] + p_ref[1] + b_ref[...]


def _combine(parts, bias, n):
    nc, _, d = parts.shape
    blk = 2000
    bias2 = bias.reshape(1, d)
    return pl.pallas_call(
        _combine_body,
        grid=(n // blk,),
        in_specs=[pl.BlockSpec((nc, blk, d), lambda i: (0, i, 0)),
                  pl.BlockSpec((1, d), lambda i: (0, 0))],
        out_specs=pl.BlockSpec((blk, d), lambda i: (i, 0)),
        out_shape=jax.ShapeDtypeStruct((n, d), jnp.float32),
    )(parts, bias2)


def _make_aggregate(n, e, d, ker):
    nw = _NC * _NS
    epw = e // nw
    chunk = 80
    nchunk = epw // chunk
    zrows = 128
    npad = ((n + zrows * _NS - 1) // (zrows * _NS)) * (zrows * _NS)
    rpt = npad // _NS

    mesh = plsc.VectorSubcoreMesh(core_axis_name="c", subcore_axis_name="s",
                                  num_cores=_NC, num_subcores=_NS)

    @functools.partial(
        pl.kernel,
        out_type=jax.ShapeDtypeStruct((_NC, npad, d), jnp.float32),
        mesh=mesh,
        scratch_types=[
            pltpu.VMEM_SHARED((npad, d), jnp.float32),
            pltpu.VMEM((chunk,), jnp.int32),
            pltpu.VMEM((chunk,), jnp.int32),
            pltpu.VMEM((chunk,), jnp.float32),
            pltpu.VMEM((chunk,), jnp.float32),
            pltpu.VMEM((chunk,), jnp.float32),
            pltpu.VMEM((chunk, ker * d), jnp.float32),
            pltpu.VMEM((chunk, d), jnp.float32),
            pltpu.SemaphoreType.DMA,
        ],
    )
    def agg(s_hbm, rows_hbm, cols_hbm, tt0_hbm, tt1_hbm, tt2_hbm, out_hbm,
            acc, cols_v, rows_v, tt0_v, tt1_v, tt2_v, g_v, msg_v, sem):
        cid = lax.axis_index("c")
        sid = lax.axis_index("s")
        wid = sid * _NC + cid

        def zfill(i, carry):
            for j in range(d // _L):
                msg_v[i, pl.ds(j * _L, _L)] = jnp.zeros((_L,), jnp.float32)
            return carry
        lax.fori_loop(0, chunk, zfill, 0)
        for z in range(rpt // chunk):
            pltpu.sync_copy(msg_v, acc.at[pl.ds(sid * rpt + z * chunk, chunk)])
        plsc.subcore_barrier()

        def chunk_body(ci, carry):
            base = wid * epw + ci * chunk
            pltpu.sync_copy(cols_hbm.at[pl.ds(base, chunk)], cols_v)
            pltpu.sync_copy(rows_hbm.at[pl.ds(base, chunk)], rows_v)
            pltpu.sync_copy(tt0_hbm.at[pl.ds(base, chunk)], tt0_v)
            pltpu.sync_copy(tt1_hbm.at[pl.ds(base, chunk)], tt1_v)
            pltpu.sync_copy(tt2_hbm.at[pl.ds(base, chunk)], tt2_v)
            pltpu.async_copy(s_hbm.at[cols_v], g_v, sem).wait()

            def group_body(g, gcarry):
                gb = g * _L
                tv0 = tt0_v[pl.ds(gb, _L)]
                tv1 = tt1_v[pl.ds(gb, _L)]
                tv2 = tt2_v[pl.ds(gb, _L)]
                for i in range(_L):
                    t0 = _splat(tv0, i)
                    t1 = _splat(tv1, i)
                    t2 = _splat(tv2, i)
                    ei = gb + i
                    for j in range(d // _L):
                        a = g_v[ei, pl.ds(j * _L, _L)]
                        b = g_v[ei, pl.ds(d + j * _L, _L)]
                        c = g_v[ei, pl.ds(2 * d + j * _L, _L)]
                        msg_v[ei, pl.ds(j * _L, _L)] = a * t0 + b * t1 + c * t2
                return gcarry
            lax.fori_loop(0, chunk // _L, group_body, 0)

            pltpu.sync_copy(msg_v, acc.at[rows_v], add=True)
            return carry
        lax.fori_loop(0, nchunk, chunk_body, 0)
        plsc.subcore_barrier()

        for z in range(rpt // zrows):
            r0 = sid * rpt + z * zrows
            pltpu.sync_copy(acc.at[pl.ds(r0, zrows)],
                            out_hbm.at[cid, pl.ds(r0, zrows)])

    return agg


def kernel(x, edge_idx, TT, weight, bias_param):
    n, d_in = x.shape
    d_out, ker = weight.shape[1], weight.shape[2]
    e = TT.shape[0]

    wf = weight.transpose(0, 2, 1).reshape(d_in, ker * d_out)
    s = _support_matmul(x, wf)

    rows = edge_idx[0]
    cols = edge_idx[1]
    tt0, tt1, tt2 = TT[:, 0], TT[:, 1], TT[:, 2]

    agg = _make_aggregate(n, e, d_out, ker)
    parts = agg(s, rows, cols, tt0, tt1, tt2)
    return _combine(parts, bias_param, n)


# speedup vs baseline: 44.6422x; 1.5543x over previous
import functools

import jax
import jax.numpy as jnp
from jax import lax
from jax.experimental import pallas as pl
from jax.experimental.pallas import tpu as pltpu
from jax.experimental.pallas import tpu_sc as plsc

_NC = 2
_NS = 16
_L = 16
_NBUF = 4

_GATHER_DNUMS = lax.GatherDimensionNumbers(
    offset_dims=(), collapsed_slice_dims=(0,), start_index_map=(0,))


def _splat(v, i):
    lane = jnp.full((_L, 1), i, jnp.int32)
    return lax.gather(v, lane, _GATHER_DNUMS, slice_sizes=(1,),
                      mode=lax.GatherScatterMode.PROMISE_IN_BOUNDS)


def _matmul_body(x_ref, w_ref, o_ref):
    o_ref[...] = jnp.dot(x_ref[...], w_ref[...],
                         preferred_element_type=jnp.float32)


def _support_matmul(x, wf):
    n, d_in = x.shape
    dk = wf.shape[1]
    blk = 2000
    return pl.pallas_call(
        _matmul_body,
        grid=(n // blk,),
        in_specs=[pl.BlockSpec((blk, d_in), lambda i: (i, 0)),
                  pl.BlockSpec((d_in, dk), lambda i: (0, 0))],
        out_specs=pl.BlockSpec((blk, dk), lambda i: (i, 0)),
        out_shape=jax.ShapeDtypeStruct((n, dk), jnp.float32),
    )(x, wf)


def _combine_body(p_ref, b_ref, o_ref):
    dh = p_ref.shape[2]
    o_ref[:, :dh] = p_ref[0] + b_ref[0, :dh]
    o_ref[:, dh:] = p_ref[1] + b_ref[0, dh:]


def _combine(parts, bias, n):
    nc, _, dh = parts.shape
    d = nc * dh
    blk = 2000
    bias2 = bias.reshape(1, d)
    return pl.pallas_call(
        _combine_body,
        grid=(n // blk,),
        in_specs=[pl.BlockSpec((nc, blk, dh), lambda i: (0, i, 0)),
                  pl.BlockSpec((1, d), lambda i: (0, 0))],
        out_specs=pl.BlockSpec((blk, d), lambda i: (i, 0)),
        out_shape=jax.ShapeDtypeStruct((n, d), jnp.float32),
    )(parts, bias2)


def _make_aggregate(n, e, d, ker):
    dh = d // _NC
    ept = e // _NS
    chunk = 80
    nchunk = ept // chunk
    zrows = 128
    npad = ((n + zrows * _NS - 1) // (zrows * _NS)) * (zrows * _NS)
    rpt = npad // _NS

    mesh = plsc.VectorSubcoreMesh(core_axis_name="c", subcore_axis_name="s",
                                  num_cores=_NC, num_subcores=_NS)

    scratch = [pltpu.VMEM_SHARED((npad, dh), jnp.float32)]
    for _ in range(_NBUF):
        scratch += [
            pltpu.VMEM((chunk,), jnp.int32),
            pltpu.VMEM((chunk,), jnp.int32),
            pltpu.VMEM((chunk,), jnp.int32),
            pltpu.VMEM((chunk,), jnp.float32),
            pltpu.VMEM((chunk,), jnp.float32),
            pltpu.VMEM((chunk,), jnp.float32),
            pltpu.VMEM((chunk, ker * dh), jnp.float32),
            pltpu.VMEM((chunk, dh), jnp.float32),
            pltpu.SemaphoreType.DMA,
            pltpu.SemaphoreType.DMA,
        ]

    @functools.partial(
        pl.kernel,
        out_type=jax.ShapeDtypeStruct((_NC, npad, dh), jnp.float32),
        mesh=mesh,
        scratch_types=scratch,
        compiler_params=pltpu.CompilerParams(use_tc_tiling_on_sc=False),
    )
    def agg(s_hbm, rows_hbm, cols_hbm, tt0_hbm, tt1_hbm, tt2_hbm, out_hbm,
            acc, *bufs):
        cid = lax.axis_index("c")
        sid = lax.axis_index("s")
        B = [bufs[i * 10:(i + 1) * 10] for i in range(_NBUF)]

        def meta_issue(b, ci):
            cols_v, _, rows_v, tt0_v, tt1_v, tt2_v, _, _, msem, _ = B[b]
            base = sid * ept + ci * chunk
            pltpu.async_copy(cols_hbm.at[pl.ds(base, chunk)], cols_v, msem)
            pltpu.async_copy(rows_hbm.at[pl.ds(base, chunk)], rows_v, msem)
            pltpu.async_copy(tt0_hbm.at[pl.ds(base, chunk)], tt0_v, msem)
            pltpu.async_copy(tt1_hbm.at[pl.ds(base, chunk)], tt1_v, msem)
            pltpu.async_copy(tt2_hbm.at[pl.ds(base, chunk)], tt2_v, msem)

        def meta_wait(b, ci):
            cols_v, _, rows_v, tt0_v, tt1_v, tt2_v, _, _, msem, _ = B[b]
            base = sid * ept + ci * chunk
            pltpu.make_async_copy(cols_hbm.at[pl.ds(base, chunk)], cols_v, msem).wait()
            pltpu.make_async_copy(rows_hbm.at[pl.ds(base, chunk)], rows_v, msem).wait()
            pltpu.make_async_copy(tt0_hbm.at[pl.ds(base, chunk)], tt0_v, msem).wait()
            pltpu.make_async_copy(tt1_hbm.at[pl.ds(base, chunk)], tt1_v, msem).wait()
            pltpu.make_async_copy(tt2_hbm.at[pl.ds(base, chunk)], tt2_v, msem).wait()

        def gather_issue(b):
            cols_v, idx_v, _, _, _, _, g_v, _, _, gsem = B[b]
            for v in range(chunk // _L):
                cv = cols_v[pl.ds(v * _L, _L)]
                idx_v[pl.ds(v * _L, _L)] = cv * 2 + cid
            pltpu.async_copy(s_hbm.at[idx_v], g_v, gsem)

        def gather_wait(b):
            _, idx_v, _, _, _, _, g_v, _, _, gsem = B[b]
            pltpu.make_async_copy(s_hbm.at[idx_v], g_v, gsem).wait()

        def compute_scatter(b):
            _, _, rows_v, tt0_v, tt1_v, tt2_v, g_v, msg_v, _, _ = B[b]

            def group_body(g, gcarry):
                gb = g * _L
                tv0 = tt0_v[pl.ds(gb, _L)]
                tv1 = tt1_v[pl.ds(gb, _L)]
                tv2 = tt2_v[pl.ds(gb, _L)]
                for i in range(_L):
                    t0 = _splat(tv0, i)
                    t1 = _splat(tv1, i)
                    t2 = _splat(tv2, i)
                    ei = gb + i
                    for j in range(dh // _L):
                        a = g_v[ei, pl.ds(j * _L, _L)]
                        bb = g_v[ei, pl.ds(dh + j * _L, _L)]
                        c = g_v[ei, pl.ds(2 * dh + j * _L, _L)]
                        msg_v[ei, pl.ds(j * _L, _L)] = a * t0 + bb * t1 + c * t2
                return gcarry
            lax.fori_loop(0, chunk // _L, group_body, 0)
            pltpu.sync_copy(msg_v, acc.at[rows_v], add=True)

        msg0 = B[0][7]
        def zfill(i, carry):
            for j in range(dh // _L):
                msg0[i, pl.ds(j * _L, _L)] = jnp.zeros((_L,), jnp.float32)
            return carry
        lax.fori_loop(0, chunk, zfill, 0)
        for z in range(rpt // chunk):
            pltpu.sync_copy(msg0, acc.at[pl.ds(sid * rpt + z * chunk, chunk)])
        plsc.subcore_barrier()

        for c in range(min(_NBUF, nchunk)):
            meta_issue(c % _NBUF, c)
        for c in range(min(2, nchunk)):
            meta_wait(c % _NBUF, c)
            gather_issue(c % _NBUF)

        def make_body():
            def body(j, carry):
                for p in range(_NBUF):
                    c = j * _NBUF + p
                    b = p
                    nb = (p + 2) % _NBUF

                    @pl.when(c + 2 < nchunk)
                    def _issue_gather():
                        meta_wait(nb, c + 2)
                        gather_issue(nb)

                    gather_wait(b)
                    compute_scatter(b)

                    @pl.when(c + _NBUF < nchunk)
                    def _issue_meta():
                        meta_issue(b, c + _NBUF)
                return carry
            return body

        nbody = nchunk // _NBUF
        lax.fori_loop(0, nbody, make_body(), 0)
        for c in range(nbody * _NBUF, nchunk):
            p = c % _NBUF
            if c + 2 < nchunk:
                meta_wait((p + 2) % _NBUF, c + 2)
                gather_issue((p + 2) % _NBUF)
            gather_wait(p)
            compute_scatter(p)

        plsc.subcore_barrier()
        for z in range(rpt // zrows):
            r0 = sid * rpt + z * zrows
            pltpu.sync_copy(acc.at[pl.ds(r0, zrows)],
                            out_hbm.at[cid, pl.ds(r0, zrows)])

    return agg


def kernel(x, edge_idx, TT, weight, bias_param):
    n, d_in = x.shape
    d_out, ker = weight.shape[1], weight.shape[2]
    e = TT.shape[0]
    dh = d_out // _NC

    w2 = (weight.reshape(d_in, _NC, dh, ker)
          .transpose(0, 1, 3, 2)
          .reshape(d_in, d_out * ker))
    s = _support_matmul(x, w2)
    s2 = s.reshape(n * _NC, ker * dh)

    rows = edge_idx[0]
    cols = edge_idx[1]
    tt0, tt1, tt2 = TT[:, 0], TT[:, 1], TT[:, 2]

    agg = _make_aggregate(n, e, d_out, ker)
    parts = agg(s2, rows, cols, tt0, tt1, tt2)
    return _combine(parts, bias_param, n)
